# baseline (speedup 1.0000x reference)
import functools

import jax
import jax.numpy as jnp
from jax import lax
from jax.experimental import pallas as pl
from jax.experimental.pallas import tpu as pltpu
from jax.experimental.pallas import tpu_sc as plsc

NC, NS, L = 2, 16, 16
NW = NC * NS
B, A, E = 16384, 20, 64
E2 = 2 * E
BW = B // NW
CB = 8
NCH = BW // CB
NPAIR = NCH // 2
PR = CB * A
CR = 2 * PR
SI = 3 * PR
GSL = 80

N_CONST_EVEN = 1000100
N_PRED_EVEN = 1000002


def _mesh():
    return plsc.VectorSubcoreMesh(
        core_axis_name="c", subcore_axis_name="s",
        num_cores=NC, num_subcores=NS,
    )


@functools.cache
def _build_const_sc():
    @functools.partial(
        pl.kernel,
        out_type=jax.ShapeDtypeStruct((B, E), jnp.float32),
        mesh=_mesh(),
        scratch_types=[
            pltpu.VMEM((SI,), jnp.int32),
            pltpu.VMEM((SI,), jnp.int32),
            pltpu.VMEM((CR,), jnp.int32),
            pltpu.VMEM((CR,), jnp.int32),
            pltpu.VMEM((CR * L,), jnp.int32),
            pltpu.VMEM((CR * L,), jnp.int32),
            pltpu.VMEM((CR, E2), jnp.float32),
            pltpu.VMEM((CR, E2), jnp.float32),
            pltpu.VMEM((BW, E), jnp.float32),
            pltpu.SemaphoreType.DMA,
            pltpu.SemaphoreType.DMA,
        ],
        compiler_params=pltpu.CompilerParams(
            use_tc_tiling_on_sc=False, needs_layout_passes=False),
    )
    def _const_sc(sub_hbm, ctab_hbm, out_hbm, sub_v0, sub_v1, cidx_v0,
                  cidx_v1, par_v0, par_v1, crow_v0, crow_v1, out_v,
                  sem0, sem1):
        wid = lax.axis_index("s") * NC + lax.axis_index("c")
        base = wid * BW
        lanes = lax.iota(jnp.int32, L)

        def stage(ch, sub_v, cidx_v, par_v, crow_v, sem):
            pltpu.sync_copy(
                sub_hbm.at[pl.ds((base + ch * CB) * (3 * A), SI)], sub_v)
            for i in range(CR // L):
                k = lanes + i * L
                src = (k >> 1) * 3 + 1 + (k & 1)
                raw = plsc.load_gather(sub_v, [src])
                cidx_v[pl.ds(i * L, L)] = raw >> 1
                plsc.store_scatter(par_v, [k * L], (raw & 1) * E)
            copies = []
            for k in range(CR // GSL):
                copies.append(pltpu.async_copy(
                    ctab_hbm.at[cidx_v.at[pl.ds(k * GSL, GSL)]],
                    crow_v.at[pl.ds(k * GSL, GSL)], sem))
            return copies

        def reduce(ch, par_v, crow_v):
            for b in range(CB):
                def atom_body(a, accs):
                    c_row = 2 * (b * A + a)
                    off_h = par_v[pl.ds(c_row * L, L)][0]
                    off_t = par_v[pl.ds((c_row + 1) * L, L)][0]
                    out = []
                    for s in range(E // L):
                        h = crow_v[c_row, pl.ds(off_h + s * L, L)]
                        t = crow_v[c_row + 1, pl.ds(off_t + s * L, L)]
                        out.append(accs[s] + (h - t))
                    return tuple(out)

                z = jnp.zeros((L,), jnp.float32)
                accs = lax.fori_loop(0, A, atom_body, (z, z, z, z))
                row = ch * CB + b
                for s in range(E // L):
                    out_v[row, pl.ds(s * L, L)] = accs[s]

        def pair_body(t, carry):
            c0 = 2 * t
            c1 = c0 + 1
            copies0 = stage(c0, sub_v0, cidx_v0, par_v0, crow_v0, sem0)
            copies1 = stage(c1, sub_v1, cidx_v1, par_v1, crow_v1, sem1)
            for cp in copies0:
                cp.wait()
            reduce(c0, par_v0, crow_v0)
            for cp in copies1:
                cp.wait()
            reduce(c1, par_v1, crow_v1)
            return carry

        lax.fori_loop(0, NPAIR, pair_body, 0)
        pltpu.sync_copy(out_v, out_hbm.at[pl.ds(base, BW)])

    return _const_sc


@functools.cache
def _build_pred_sc():
    @functools.partial(
        pl.kernel,
        out_type=jax.ShapeDtypeStruct((B, E), jnp.float32),
        mesh=_mesh(),
        scratch_types=[
            pltpu.VMEM((SI,), jnp.int32),
            pltpu.VMEM((SI,), jnp.int32),
            pltpu.VMEM((PR,), jnp.int32),
            pltpu.VMEM((PR,), jnp.int32),
            pltpu.VMEM((PR * L,), jnp.int32),
            pltpu.VMEM((PR * L,), jnp.int32),
            pltpu.VMEM((PR, E2), jnp.float32),
            pltpu.VMEM((PR, E2), jnp.float32),
            pltpu.VMEM((BW, E), jnp.float32),
            pltpu.SemaphoreType.DMA,
            pltpu.SemaphoreType.DMA,
        ],
        compiler_params=pltpu.CompilerParams(
            use_tc_tiling_on_sc=False, needs_layout_passes=False),
    )
    def _pred_sc(sub_hbm, ptab_hbm, csum_hbm, out_hbm, sub_v0, sub_v1,
                 pidx_v0, pidx_v1, par_v0, par_v1, prow_v0, prow_v1, out_v,
                 sem0, sem1):
        wid = lax.axis_index("s") * NC + lax.axis_index("c")
        base = wid * BW
        lanes = lax.iota(jnp.int32, L)

        pltpu.sync_copy(csum_hbm.at[pl.ds(base, BW)], out_v)

        def stage(ch, sub_v, pidx_v, par_v, prow_v, sem):
            pltpu.sync_copy(
                sub_hbm.at[pl.ds((base + ch * CB) * (3 * A), SI)], sub_v)
            for i in range(PR // L):
                src = lanes * 3 + (i * 3 * L)
                raw = plsc.load_gather(sub_v, [src])
                pidx_v[pl.ds(i * L, L)] = raw >> 1
                plsc.store_scatter(par_v, [(lanes + i * L) * L],
                                   (raw & 1) * E)
            copies = []
            for k in range(PR // GSL):
                copies.append(pltpu.async_copy(
                    ptab_hbm.at[pidx_v.at[pl.ds(k * GSL, GSL)]],
                    prow_v.at[pl.ds(k * GSL, GSL)], sem))
            return copies

        def reduce(ch, par_v, prow_v):
            for b in range(CB):
                def atom_body(a, accs):
                    p_row = b * A + a
                    off_p = par_v[pl.ds(p_row * L, L)][0]
                    out = []
                    for s in range(E // L):
                        out.append(
                            accs[s] + prow_v[p_row, pl.ds(off_p + s * L, L)])
                    return tuple(out)

                row = ch * CB + b
                init = tuple(out_v[row, pl.ds(s * L, L)]
                             for s in range(E // L))
                accs = lax.fori_loop(0, A, atom_body, init)
                for s in range(E // L):
                    out_v[row, pl.ds(s * L, L)] = accs[s]

        def pair_body(t, carry):
            c0 = 2 * t
            c1 = c0 + 1
            copies0 = stage(c0, sub_v0, pidx_v0, par_v0, prow_v0, sem0)
            copies1 = stage(c1, sub_v1, pidx_v1, par_v1, prow_v1, sem1)
            for cp in copies0:
                cp.wait()
            reduce(c0, par_v0, prow_v0)
            for cp in copies1:
                cp.wait()
            reduce(c1, par_v1, prow_v1)
            return carry

        lax.fori_loop(0, NPAIR, pair_body, 0)
        pltpu.sync_copy(out_v, out_hbm.at[pl.ds(base, BW)])

    return _pred_sc


def kernel(sub_indices, const_table, pred_table):
    sub_flat = sub_indices.astype(jnp.int32).reshape(B * A * 3)
    ctab_pk = jnp.concatenate(
        [const_table[0:N_CONST_EVEN:2], const_table[1:N_CONST_EVEN:2]],
        axis=1)
    ptab_pk = jnp.concatenate(
        [pred_table[0:N_PRED_EVEN:2], pred_table[1:N_PRED_EVEN:2]],
        axis=1)
    csum = _build_const_sc()(sub_flat, ctab_pk)
    return _build_pred_sc()(sub_flat, ptab_pk, csum)


# speedup vs baseline: 13.5448x; 13.5448x over previous
import functools

import jax
import jax.numpy as jnp
from jax import lax
from jax.experimental import pallas as pl
from jax.experimental.pallas import tpu as pltpu
from jax.experimental.pallas import tpu_sc as plsc

NC, NS, L = 2, 16, 16
NW = NC * NS
B, A, E = 16384, 20, 64
BW = B // NW
CB = 16
NCH = BW // CB
NPAIR = NCH // 2
PR = CB * A
CR = 2 * PR
SI = 3 * PR
GSL = 80
GSLC = 128


def _mesh():
    return plsc.VectorSubcoreMesh(
        core_axis_name="c", subcore_axis_name="s",
        num_cores=NC, num_subcores=NS,
    )


@functools.cache
def _build_const_sc():
    @functools.partial(
        pl.kernel,
        out_type=jax.ShapeDtypeStruct((B, E), jnp.float32),
        mesh=_mesh(),
        scratch_types=[
            pltpu.VMEM((SI,), jnp.int32),
            pltpu.VMEM((SI,), jnp.int32),
            pltpu.VMEM((CR,), jnp.int32),
            pltpu.VMEM((CR,), jnp.int32),
            pltpu.VMEM((CR, E), jnp.float32),
            pltpu.VMEM((CR, E), jnp.float32),
            pltpu.VMEM((BW, E), jnp.float32),
            pltpu.SemaphoreType.DMA,
            pltpu.SemaphoreType.DMA,
        ],
        compiler_params=pltpu.CompilerParams(
            use_tc_tiling_on_sc=False, needs_layout_passes=False),
    )
    def _const_sc(sub_hbm, ctab_hbm, out_hbm, sub_v0, sub_v1, cidx_v0,
                  cidx_v1, crow_v0, crow_v1, out_v, sem0, sem1):
        wid = lax.axis_index("s") * NC + lax.axis_index("c")
        base = wid * BW
        lanes = lax.iota(jnp.int32, L)

        def stage(ch, sub_v, cidx_v, crow_v, sem):
            pltpu.sync_copy(
                sub_hbm.at[pl.ds((base + ch * CB) * (3 * A), SI)], sub_v)
            for i in range(CR // L):
                k = lanes + i * L
                src = (k >> 1) * 3 + 1 + (k & 1)
                cidx_v[pl.ds(i * L, L)] = plsc.load_gather(sub_v, [src])
            copies = []
            for k in range(CR // GSLC):
                copies.append(pltpu.async_copy(
                    ctab_hbm.at[cidx_v.at[pl.ds(k * GSLC, GSLC)]],
                    crow_v.at[pl.ds(k * GSLC, GSLC)], sem))
            return copies

        def reduce(ch, crow_v):
            for b in range(CB):
                def atom_body(a, accs):
                    c_row = 2 * (b * A + a)
                    out = []
                    for s in range(E // L):
                        sl = pl.ds(s * L, L)
                        h = crow_v[c_row, sl]
                        t = crow_v[c_row + 1, sl]
                        out.append(accs[s] + (h - t))
                    return tuple(out)

                z = jnp.zeros((L,), jnp.float32)
                accs = lax.fori_loop(0, A, atom_body, (z, z, z, z))
                row = ch * CB + b
                for s in range(E // L):
                    out_v[row, pl.ds(s * L, L)] = accs[s]

        def pair_body(t, carry):
            c0 = 2 * t
            c1 = c0 + 1
            copies0 = stage(c0, sub_v0, cidx_v0, crow_v0, sem0)
            copies1 = stage(c1, sub_v1, cidx_v1, crow_v1, sem1)
            for cp in copies0:
                cp.wait()
            reduce(c0, crow_v0)
            for cp in copies1:
                cp.wait()
            reduce(c1, crow_v1)
            return carry

        lax.fori_loop(0, NPAIR, pair_body, 0)
        pltpu.sync_copy(out_v, out_hbm.at[pl.ds(base, BW)])

    return _const_sc


@functools.cache
def _build_pred_sc():
    @functools.partial(
        pl.kernel,
        out_type=jax.ShapeDtypeStruct((B, E), jnp.float32),
        mesh=_mesh(),
        scratch_types=[
            pltpu.VMEM((SI,), jnp.int32),
            pltpu.VMEM((SI,), jnp.int32),
            pltpu.VMEM((PR,), jnp.int32),
            pltpu.VMEM((PR,), jnp.int32),
            pltpu.VMEM((PR, E), jnp.float32),
            pltpu.VMEM((PR, E), jnp.float32),
            pltpu.VMEM((BW, E), jnp.float32),
            pltpu.SemaphoreType.DMA,
            pltpu.SemaphoreType.DMA,
        ],
        compiler_params=pltpu.CompilerParams(
            use_tc_tiling_on_sc=False, needs_layout_passes=False),
    )
    def _pred_sc(sub_hbm, ptab_hbm, csum_hbm, out_hbm, sub_v0, sub_v1,
                 pidx_v0, pidx_v1, prow_v0, prow_v1, out_v, sem0, sem1):
        wid = lax.axis_index("s") * NC + lax.axis_index("c")
        base = wid * BW
        lanes = lax.iota(jnp.int32, L)

        pltpu.sync_copy(csum_hbm.at[pl.ds(base, BW)], out_v)

        def stage(ch, sub_v, pidx_v, prow_v, sem):
            pltpu.sync_copy(
                sub_hbm.at[pl.ds((base + ch * CB) * (3 * A), SI)], sub_v)
            for i in range(PR // L):
                src = lanes * 3 + (i * 3 * L)
                pidx_v[pl.ds(i * L, L)] = plsc.load_gather(sub_v, [src])
            copies = []
            for k in range(PR // GSL):
                copies.append(pltpu.async_copy(
                    ptab_hbm.at[pidx_v.at[pl.ds(k * GSL, GSL)]],
                    prow_v.at[pl.ds(k * GSL, GSL)], sem))
            return copies

        def reduce(ch, prow_v):
            for b in range(CB):
                def atom_body(a, accs):
                    p_row = b * A + a
                    out = []
                    for s in range(E // L):
                        sl = pl.ds(s * L, L)
                        out.append(accs[s] + prow_v[p_row, sl])
                    return tuple(out)

                row = ch * CB + b
                init = tuple(out_v[row, pl.ds(s * L, L)]
                             for s in range(E // L))
                accs = lax.fori_loop(0, A, atom_body, init)
                for s in range(E // L):
                    out_v[row, pl.ds(s * L, L)] = accs[s]

        def pair_body(t, carry):
            c0 = 2 * t
            c1 = c0 + 1
            copies0 = stage(c0, sub_v0, pidx_v0, prow_v0, sem0)
            copies1 = stage(c1, sub_v1, pidx_v1, prow_v1, sem1)
            for cp in copies0:
                cp.wait()
            reduce(c0, prow_v0)
            for cp in copies1:
                cp.wait()
            reduce(c1, prow_v1)
            return carry

        lax.fori_loop(0, NPAIR, pair_body, 0)
        pltpu.sync_copy(out_v, out_hbm.at[pl.ds(base, BW)])

    return _pred_sc


def kernel(sub_indices, const_table, pred_table):
    sub_flat = sub_indices.astype(jnp.int32).reshape(B * A * 3)
    csum = _build_const_sc()(sub_flat, const_table)
    return _build_pred_sc()(sub_flat, pred_table, csum)
